# Initial kernel scaffold; baseline (speedup 1.0000x reference)
import jax
import jax.numpy as jnp
from jax.experimental import pallas as pl


def kernel(inputs, pos_table):
    raise NotImplementedError("write your pallas kernel here")


# speedup vs baseline: 1.8253x; 1.8253x over previous
import functools

import jax
import jax.numpy as jnp
from jax import lax
from jax.experimental import pallas as pl
from jax.experimental.pallas import tpu as pltpu
from jax.experimental.pallas import tpu_sc as plsc

_NC = 2
_NS = 16
_NW = _NC * _NS
_LANES = 16


def _build_sc_gather(n_rows, seq_len, d_model):
    rows_per_w = n_rows // _NW
    chunk = 32
    n_chunks = rows_per_w // chunk
    mesh = plsc.VectorSubcoreMesh(core_axis_name="c", subcore_axis_name="s")

    @functools.partial(
        pl.kernel,
        out_type=jax.ShapeDtypeStruct((n_rows, d_model), jnp.float32),
        mesh=mesh,
        scratch_types=[
            pltpu.VMEM((rows_per_w,), jnp.int32),
            pltpu.VMEM((n_chunks, chunk), jnp.int32),
            pltpu.VMEM((chunk, d_model), jnp.float32),
            pltpu.VMEM((chunk, d_model), jnp.float32),
            pltpu.SemaphoreType.DMA,
            pltpu.SemaphoreType.DMA,
            pltpu.SemaphoreType.DMA,
            pltpu.SemaphoreType.DMA,
        ],
    )
    def k(table_hbm, tok_hbm, out_hbm, tok_v, idx_v, buf_a, buf_b,
          gsem_a, gsem_b, wsem_a, wsem_b):
        wid = lax.axis_index("s") * _NC + lax.axis_index("c")
        base = wid * rows_per_w
        s0 = lax.rem(base, seq_len)

        pltpu.sync_copy(tok_hbm.at[pl.ds(base, rows_per_w)], tok_v)

        for i in range(rows_per_w // _LANES):
            tok = tok_v[pl.ds(i * _LANES, _LANES)]
            pos = (s0 + (i * _LANES + 1)) + lax.iota(jnp.int32, 16)
            idx = jnp.where(tok == 0, 0, pos)
            c = (i * _LANES) // chunk
            off = (i * _LANES) % chunk
            idx_v[c, pl.ds(off, _LANES)] = idx

        bufs = (buf_a, buf_b)
        gsems = (gsem_a, gsem_b)
        wsems = (wsem_a, wsem_b)
        gathers = [None] * n_chunks
        writes = [None] * n_chunks
        gathers[0] = pltpu.async_copy(
            table_hbm.at[idx_v.at[0]], bufs[0], gsems[0])
        for c in range(n_chunks):
            b = c & 1
            gathers[c].wait()
            if c >= 1:
                writes[c - 1].wait()
            if c + 1 < n_chunks:
                gathers[c + 1] = pltpu.async_copy(
                    table_hbm.at[idx_v.at[c + 1]], bufs[1 - b], gsems[1 - b])
            writes[c] = pltpu.async_copy(
                bufs[b], out_hbm.at[pl.ds(base + c * chunk, chunk)], wsems[b])
        writes[n_chunks - 1].wait()

    return k


def kernel(inputs, pos_table):
    batch, seq_len = inputs.shape
    d_model = pos_table.shape[1]
    n_rows = batch * seq_len
    tok = inputs.reshape(n_rows).astype(jnp.int32)
    k = _build_sc_gather(n_rows, seq_len, d_model)
    out = k(pos_table, tok)
    return out.reshape(batch, seq_len, d_model)


# speedup vs baseline: 2.4055x; 1.3178x over previous
import functools

import jax
import jax.numpy as jnp
from jax import lax
from jax.experimental import pallas as pl
from jax.experimental.pallas import tpu as pltpu
from jax.experimental.pallas import tpu_sc as plsc

_NC = 2
_NS = 16
_NW = _NC * _NS
_LANES = 16


def _build_sc_posenc(batch, seq_len, d_model):
    s_per_w = seq_len // _NW
    chunk = 32
    n_chunks = s_per_w // chunk
    mesh = plsc.VectorSubcoreMesh(core_axis_name="c", subcore_axis_name="s")

    @functools.partial(
        pl.kernel,
        out_type=jax.ShapeDtypeStruct((batch * seq_len, d_model), jnp.float32),
        mesh=mesh,
        scratch_types=[
            pltpu.VMEM((batch, s_per_w), jnp.int32),
            pltpu.VMEM((s_per_w // 32, chunk), jnp.int32),
            pltpu.VMEM((chunk, d_model), jnp.float32),
            pltpu.VMEM((chunk, d_model), jnp.float32),
            pltpu.VMEM((chunk, d_model), jnp.float32),
            pltpu.VMEM((chunk,), jnp.int32),
            pltpu.SemaphoreType.DMA,
            pltpu.SemaphoreType.DMA,
            pltpu.SemaphoreType.DMA,
            pltpu.SemaphoreType.DMA,
        ],
    )
    def k(table_hbm, tok_hbm, out_hbm, tok_v, pidx_v, buf_a, buf_b, fix_v,
          fidx_v, rsem_a, rsem_b, wsem_a, wsem_b):
        wid = lax.axis_index("s") * _NC + lax.axis_index("c")
        s0 = wid * s_per_w

        for b in range(batch):
            pltpu.sync_copy(tok_hbm.at[b, pl.ds(s0, s_per_w)], tok_v.at[b])

        for i in range(s_per_w // _LANES):
            pos = (s0 + (i * _LANES + 1)) + lax.iota(jnp.int32, 16)
            pidx_v[(i * _LANES) // chunk,
                   pl.ds((i * _LANES) % chunk, _LANES)] = pos

        bufs = (buf_a, buf_b)
        rsems = (rsem_a, rsem_b)
        wsems = (wsem_a, wsem_b)
        reads = [None] * n_chunks
        writes = [[None] * batch for _ in range(n_chunks)]

        def read_chunk(c, slot):
            return pltpu.async_copy(
                table_hbm.at[pidx_v.at[c]], bufs[slot], rsems[slot])

        reads[0] = read_chunk(0, 0)
        for c in range(n_chunks):
            slot = c & 1
            reads[c].wait()
            if c >= 1:
                for w in writes[c - 1]:
                    w.wait()
            if c + 1 < n_chunks:
                reads[c + 1] = read_chunk(c + 1, 1 - slot)
            for b in range(batch):
                writes[c][b] = pltpu.async_copy(
                    bufs[slot],
                    out_hbm.at[pl.ds(b * seq_len + s0 + c * chunk, chunk)],
                    wsems[slot])
        for w in writes[n_chunks - 1]:
            w.wait()

        def any_zero_scalar(vreg):
            anyz = None
            for j in range(_LANES):
                lz = vreg[j] == 0
                anyz = lz if anyz is None else (anyz | lz)
            return anyz

        mv = None
        for b in range(batch):
            for i in range(s_per_w // _LANES):
                t = tok_v[b, pl.ds(i * _LANES, _LANES)]
                mv = t if mv is None else jnp.minimum(mv, t)

        @pl.when(any_zero_scalar(mv))
        def _fixups():
            for c in range(n_chunks):
                for b in range(batch):
                    t1 = tok_v[b, pl.ds(c * chunk, _LANES)]
                    t2 = tok_v[b, pl.ds(c * chunk + _LANES, _LANES)]

                    @pl.when(any_zero_scalar(jnp.minimum(t1, t2)))
                    def _fix(c=c, b=b):
                        for i in range(chunk // _LANES):
                            tok = tok_v[b, pl.ds(c * chunk + i * _LANES,
                                                 _LANES)]
                            pos = (s0 + (c * chunk + i * _LANES + 1)
                                   ) + lax.iota(jnp.int32, 16)
                            fidx_v[pl.ds(i * _LANES, _LANES)] = jnp.where(
                                tok == 0, 0, pos)
                        pltpu.async_copy(
                            table_hbm.at[fidx_v], fix_v, rsems[0]).wait()
                        pltpu.async_copy(
                            fix_v,
                            out_hbm.at[
                                pl.ds(b * seq_len + s0 + c * chunk, chunk)],
                            wsems[0]).wait()

    return k


def kernel(inputs, pos_table):
    batch, seq_len = inputs.shape
    d_model = pos_table.shape[1]
    tok = inputs.astype(jnp.int32)
    k = _build_sc_posenc(batch, seq_len, d_model)
    out = k(pos_table, tok)
    return out.reshape(batch, seq_len, d_model)


# speedup vs baseline: 2.4992x; 1.0389x over previous
import functools

import jax
import jax.numpy as jnp
from jax import lax
from jax.experimental import pallas as pl
from jax.experimental.pallas import tpu as pltpu
from jax.experimental.pallas import tpu_sc as plsc

_NC = 2
_NS = 16
_NW = _NC * _NS
_LANES = 16


def _build_sc_posenc(batch, seq_len, d_model):
    s_per_w = seq_len // _NW
    chunk = 32
    n_chunks = s_per_w // chunk
    nbuf = 3
    mesh = plsc.VectorSubcoreMesh(core_axis_name="c", subcore_axis_name="s")

    @functools.partial(
        pl.kernel,
        out_type=jax.ShapeDtypeStruct((batch * seq_len, d_model), jnp.float32),
        mesh=mesh,
        scratch_types=[
            pltpu.VMEM((batch, s_per_w), jnp.int32),
            pltpu.VMEM((s_per_w // 32, chunk), jnp.int32),
            pltpu.VMEM((chunk, d_model), jnp.float32),
            pltpu.VMEM((chunk, d_model), jnp.float32),
            pltpu.VMEM((chunk, d_model), jnp.float32),
            pltpu.VMEM((chunk,), jnp.int32),
            pltpu.SemaphoreType.DMA,
            pltpu.SemaphoreType.DMA,
            pltpu.SemaphoreType.DMA,
            pltpu.SemaphoreType.DMA,
            pltpu.SemaphoreType.DMA,
            pltpu.SemaphoreType.DMA,
            pltpu.SemaphoreType.DMA,
        ],
    )
    def k(table_hbm, tok_hbm, out_hbm, tok_v, pidx_v, buf0, buf1, buf2,
          fidx_v, rsem0, rsem1, rsem2, wsem0, wsem1, wsem2, tsem):
        wid = lax.axis_index("s") * _NC + lax.axis_index("c")
        s0 = wid * s_per_w

        for i in range(s_per_w // _LANES):
            pos = (s0 + (i * _LANES + 1)) + lax.iota(jnp.int32, 16)
            pidx_v[(i * _LANES) // chunk,
                   pl.ds((i * _LANES) % chunk, _LANES)] = pos

        bufs = (buf0, buf1, buf2)
        rsems = (rsem0, rsem1, rsem2)
        wsems = (wsem0, wsem1, wsem2)

        def read_chunk(c):
            return pltpu.async_copy(
                table_hbm.at[pidx_v.at[c]], bufs[c % nbuf], rsems[c % nbuf])

        reads = [None] * n_chunks
        for c in range(min(nbuf, n_chunks)):
            reads[c] = read_chunk(c)
        tok_copies = [
            pltpu.async_copy(tok_hbm.at[b, pl.ds(s0, s_per_w)],
                             tok_v.at[b], tsem)
            for b in range(batch)
        ]

        writes = [[None] * batch for _ in range(n_chunks)]
        for c in range(n_chunks):
            reads[c].wait()
            if c == n_chunks - 2 and n_chunks > nbuf:
                for w in writes[0]:
                    w.wait()
                reads[n_chunks - 1] = read_chunk(n_chunks - 1)
            for b in range(batch):
                writes[c][b] = pltpu.async_copy(
                    bufs[c % nbuf],
                    out_hbm.at[pl.ds(b * seq_len + s0 + c * chunk, chunk)],
                    wsems[c % nbuf])
        for c in range(1 if n_chunks > nbuf else 0, n_chunks):
            for w in writes[c]:
                w.wait()
        for t in tok_copies:
            t.wait()

        def any_zero_scalar(vreg):
            anyz = None
            for j in range(_LANES):
                lz = vreg[j] == 0
                anyz = lz if anyz is None else (anyz | lz)
            return anyz

        mv = None
        for b in range(batch):
            for i in range(s_per_w // _LANES):
                t = tok_v[b, pl.ds(i * _LANES, _LANES)]
                mv = t if mv is None else jnp.minimum(mv, t)

        @pl.when(any_zero_scalar(mv))
        def _fixups():
            for c in range(n_chunks):
                for b in range(batch):
                    t1 = tok_v[b, pl.ds(c * chunk, _LANES)]
                    t2 = tok_v[b, pl.ds(c * chunk + _LANES, _LANES)]

                    @pl.when(any_zero_scalar(jnp.minimum(t1, t2)))
                    def _fix(c=c, b=b):
                        for i in range(chunk // _LANES):
                            tok = tok_v[b, pl.ds(c * chunk + i * _LANES,
                                                 _LANES)]
                            pos = (s0 + (c * chunk + i * _LANES + 1)
                                   ) + lax.iota(jnp.int32, 16)
                            fidx_v[pl.ds(i * _LANES, _LANES)] = jnp.where(
                                tok == 0, 0, pos)
                        pltpu.async_copy(
                            table_hbm.at[fidx_v], buf0, rsems[0]).wait()
                        pltpu.async_copy(
                            buf0,
                            out_hbm.at[
                                pl.ds(b * seq_len + s0 + c * chunk, chunk)],
                            wsems[0]).wait()

    return k


def kernel(inputs, pos_table):
    batch, seq_len = inputs.shape
    d_model = pos_table.shape[1]
    tok = inputs.astype(jnp.int32)
    k = _build_sc_posenc(batch, seq_len, d_model)
    out = k(pos_table, tok)
    return out.reshape(batch, seq_len, d_model)


# speedup vs baseline: 2.7160x; 1.0868x over previous
import functools

import jax
import jax.numpy as jnp
from jax import lax
from jax.experimental import pallas as pl
from jax.experimental.pallas import tpu as pltpu
from jax.experimental.pallas import tpu_sc as plsc

_NC = 2
_NS = 16
_NW = _NC * _NS
_LANES = 16


def _build_sc_posenc(batch, seq_len, d_model):
    s_per_w = seq_len // _NW
    chunk = 32
    n_chunks = s_per_w // chunk
    nbuf = 3
    mesh = plsc.VectorSubcoreMesh(core_axis_name="c", subcore_axis_name="s")

    @functools.partial(
        pl.kernel,
        out_type=jax.ShapeDtypeStruct((batch * seq_len, d_model), jnp.float32),
        mesh=mesh,
        scratch_types=[
            pltpu.VMEM((batch, s_per_w), jnp.int32),
            pltpu.VMEM((s_per_w // 32, chunk), jnp.int32),
            pltpu.VMEM((chunk, d_model), jnp.float32),
            pltpu.VMEM((chunk, d_model), jnp.float32),
            pltpu.VMEM((chunk, d_model), jnp.float32),
            pltpu.VMEM((chunk,), jnp.int32),
            pltpu.SemaphoreType.DMA,
            pltpu.SemaphoreType.DMA,
            pltpu.SemaphoreType.DMA,
            pltpu.SemaphoreType.DMA,
            pltpu.SemaphoreType.DMA,
            pltpu.SemaphoreType.DMA,
            pltpu.SemaphoreType.DMA,
        ],
    )
    def k(table_hbm, tok_hbm, out_hbm, tok_v, pidx_v, buf0, buf1, buf2,
          fidx_v, rsem0, rsem1, rsem2, wsem0, wsem1, wsem2, tsem):
        wid = lax.axis_index("s") * _NC + lax.axis_index("c")
        s0 = wid * s_per_w

        for i in range(s_per_w // _LANES):
            pos = (s0 + (i * _LANES + 1)) + lax.iota(jnp.int32, 16)
            pidx_v[(i * _LANES) // chunk,
                   pl.ds((i * _LANES) % chunk, _LANES)] = pos

        bufs = (buf0, buf1, buf2)
        rsems = (rsem0, rsem1, rsem2)
        wsems = (wsem0, wsem1, wsem2)

        def read_chunk(c):
            return pltpu.async_copy(
                table_hbm.at[pidx_v.at[c]], bufs[c % nbuf], rsems[c % nbuf])

        reads = [None] * n_chunks
        for c in range(min(nbuf, n_chunks)):
            reads[c] = read_chunk(c)
        tok_copies = [
            pltpu.async_copy(tok_hbm.at[b, pl.ds(s0, s_per_w)],
                             tok_v.at[b], tsem)
            for b in range(batch)
        ]

        writes = [[None] * batch for _ in range(n_chunks)]
        for c in range(n_chunks):
            reads[c].wait()
            if c == n_chunks - 2 and n_chunks > nbuf:
                for w in writes[0]:
                    w.wait()
                reads[n_chunks - 1] = read_chunk(n_chunks - 1)
            for b in range(batch):
                writes[c][b] = pltpu.async_copy(
                    bufs[c % nbuf],
                    out_hbm.at[pl.ds(b * seq_len + s0 + c * chunk, chunk)],
                    wsems[c % nbuf])
        for c in range(1 if n_chunks > nbuf else 0, n_chunks):
            for w in writes[c]:
                w.wait()
        for t in tok_copies:
            t.wait()

        def any_zero_scalar(vreg):
            anyz = None
            for j in range(_LANES):
                lz = vreg[j] == 0
                anyz = lz if anyz is None else (anyz | lz)
            return anyz

        def fix_span(cb, carry):
            c = cb // batch
            b = cb % batch
            t1 = tok_v[b, pl.ds(c * chunk, _LANES)]
            t2 = tok_v[b, pl.ds(c * chunk + _LANES, _LANES)]

            @pl.when(any_zero_scalar(jnp.minimum(t1, t2)))
            def _fix():
                for i in range(chunk // _LANES):
                    tok = tok_v[b, pl.ds(c * chunk + i * _LANES, _LANES)]
                    pos = (s0 + (c * chunk + i * _LANES + 1)
                           ) + lax.iota(jnp.int32, 16)
                    fidx_v[pl.ds(i * _LANES, _LANES)] = jnp.where(
                        tok == 0, 0, pos)
                pltpu.async_copy(
                    table_hbm.at[fidx_v], buf0, rsems[0]).wait()
                pltpu.async_copy(
                    buf0,
                    out_hbm.at[pl.ds(b * seq_len + s0 + c * chunk, chunk)],
                    wsems[0]).wait()

            return carry

        lax.fori_loop(0, n_chunks * batch, fix_span, 0)

    return k


def kernel(inputs, pos_table):
    batch, seq_len = inputs.shape
    d_model = pos_table.shape[1]
    tok = inputs.astype(jnp.int32)
    k = _build_sc_posenc(batch, seq_len, d_model)
    out = k(pos_table, tok)
    return out.reshape(batch, seq_len, d_model)
